# baseline (speedup 1.0000x reference)
import functools

import jax
import jax.numpy as jnp
from jax import lax
from jax.experimental import pallas as pl
from jax.experimental.pallas import tpu as pltpu
from jax.experimental.pallas import tpu_sc as plsc

_INFO = plsc.get_sparse_core_info()
_NC = _INFO.num_cores
_NS = _INFO.num_subcores
_NW = _NC * _NS

_CHUNK = 128
_NBUF = 10


def _make_gather(total, dim):
    per_w = total // _NW
    nch = per_w // _CHUNK
    ngrp = nch // _NBUF
    nsup = ngrp // 2
    assert nch % (2 * _NBUF) == 0
    mesh = plsc.VectorSubcoreMesh(core_axis_name="c", subcore_axis_name="s")

    @functools.partial(
        pl.kernel,
        out_type=jax.ShapeDtypeStruct((_NW, nch, _CHUNK, dim), jnp.float32),
        mesh=mesh,
        scratch_types=[
            pltpu.VMEM((nch, _CHUNK), jnp.int32),
            pltpu.VMEM((2, _NBUF, _CHUNK, dim), jnp.float32),
            pltpu.SemaphoreType.DMA((2, _NBUF)),
            pltpu.SemaphoreType.DMA((2,)),
        ],
        compiler_params=pltpu.CompilerParams(use_tc_tiling_on_sc=False),
    )
    def k(table_hbm, idx_hbm, out_hbm, idx_v, rows_v, gsem, ssem):
        wid = lax.axis_index("s") * _NC + lax.axis_index("c")
        pltpu.sync_copy(idx_hbm.at[wid], idx_v)

        def gather_start(j, p, b):
            pltpu.async_copy(
                table_hbm.at[idx_v.at[j]], rows_v.at[p, b], gsem.at[p, b]
            )

        def gather_wait(p, b):
            pltpu.make_async_copy(
                table_hbm.at[pl.ds(0, _CHUNK)], rows_v.at[p, b], gsem.at[p, b]
            ).wait()

        def scatter_start_group(g, p):
            pltpu.async_copy(
                rows_v.at[p], out_hbm.at[wid, pl.ds(g * _NBUF, _NBUF)], ssem.at[p]
            )

        def scatter_wait_group(p):
            pltpu.make_async_copy(
                rows_v.at[p], out_hbm.at[wid, pl.ds(0, _NBUF)], ssem.at[p]
            ).wait()

        for b in range(_NBUF):
            gather_start(b, 0, b)

        def superstep(s, carry):
            for p in range(2):
                g = 2 * s + p
                for b in range(_NBUF):
                    gather_wait(p, b)
                scatter_start_group(g, p)

                @pl.when(g + 1 < ngrp)
                def _():
                    @pl.when(g > 0)
                    def _():
                        scatter_wait_group(1 - p)
                    for b in range(_NBUF):
                        gather_start((g + 1) * _NBUF + b, 1 - p, b)
            return carry

        lax.fori_loop(0, nsup, superstep, 0)

        scatter_wait_group(0)
        scatter_wait_group(1)

    return k


def kernel(inputs, table):
    b, h = inputs.shape
    dim = table.shape[1]
    total = b * h
    idx = inputs.reshape(total).astype(jnp.int32)
    idx3 = idx.reshape(_NW, total // (_NW * _CHUNK), _CHUNK)
    out = _make_gather(total, dim)(table, idx3)
    return out.reshape(b, h, dim)


# speedup vs baseline: 1.0021x; 1.0021x over previous
import functools

import jax
import jax.numpy as jnp
from jax import lax
from jax.experimental import pallas as pl
from jax.experimental.pallas import tpu as pltpu
from jax.experimental.pallas import tpu_sc as plsc

_INFO = plsc.get_sparse_core_info()
_NC = _INFO.num_cores
_NS = _INFO.num_subcores
_NW = _NC * _NS

_CHUNK = 128
_NBUF = 5


def _make_gather(total, dim):
    per_w = total // _NW
    nch = per_w // _CHUNK
    ngrp = nch // _NBUF
    nsup = ngrp // 2
    assert nch % (2 * _NBUF) == 0
    mesh = plsc.VectorSubcoreMesh(core_axis_name="c", subcore_axis_name="s")

    @functools.partial(
        pl.kernel,
        out_type=jax.ShapeDtypeStruct((_NW, nch, _CHUNK, dim), jnp.float32),
        mesh=mesh,
        scratch_types=[
            pltpu.VMEM((nch, _CHUNK), jnp.int32),
            pltpu.VMEM((2, _NBUF, _CHUNK, dim), jnp.float32),
            pltpu.SemaphoreType.DMA((2, _NBUF)),
            pltpu.SemaphoreType.DMA((2, _NBUF)),
        ],
        compiler_params=pltpu.CompilerParams(use_tc_tiling_on_sc=False),
    )
    def k(table_hbm, idx_hbm, out_hbm, idx_v, rows_v, gsem, ssem):
        wid = lax.axis_index("s") * _NC + lax.axis_index("c")
        pltpu.sync_copy(idx_hbm.at[wid], idx_v)

        def gather_start(j, p, b):
            pltpu.async_copy(
                table_hbm.at[idx_v.at[j]], rows_v.at[p, b], gsem.at[p, b]
            )

        def gather_wait(p, b):
            pltpu.make_async_copy(
                table_hbm.at[pl.ds(0, _CHUNK)], rows_v.at[p, b], gsem.at[p, b]
            ).wait()

        def scatter_start(j, p, b):
            pltpu.async_copy(
                rows_v.at[p, b], out_hbm.at[wid, j], ssem.at[p, b]
            )

        def scatter_wait(p, b):
            pltpu.make_async_copy(
                table_hbm.at[pl.ds(0, _CHUNK)], rows_v.at[p, b], ssem.at[p, b]
            ).wait()

        for b in range(_NBUF):
            gather_start(b, 0, b)

        def superstep(s, carry):
            for p in range(2):
                g = 2 * s + p
                for b in range(_NBUF):
                    j = g * _NBUF + b
                    gather_wait(p, b)
                    scatter_start(j, p, b)
                    jn = j + _NBUF
                    @pl.when(jn < nch)
                    def _():
                        @pl.when(g > 0)
                        def _():
                            scatter_wait(1 - p, b)
                        gather_start(jn, 1 - p, b)
            return carry

        lax.fori_loop(0, nsup, superstep, 0)

        for p in range(2):
            for b in range(_NBUF):
                scatter_wait(p, b)

    return k


def kernel(inputs, table):
    b, h = inputs.shape
    dim = table.shape[1]
    total = b * h
    idx = inputs.reshape(total).astype(jnp.int32)
    idx3 = idx.reshape(_NW, total // (_NW * _CHUNK), _CHUNK)
    out = _make_gather(total, dim)(table, idx3)
    return out.reshape(b, h, dim)
